# baseline (speedup 1.0000x reference)
import functools

import jax
import jax.numpy as jnp
from jax import lax
from jax.experimental import pallas as pl
from jax.experimental.pallas import tpu as pltpu
from jax.experimental.pallas import tpu_sc as plsc

_NUM_EMB = 1024
_DIM = 64
_B = 64
_T = 576
_N = _B * _T


def _argmin_body(xt_ref, emb_ref, idx_ref):
    xtb = xt_ref[:]
    emb = emb_ref[:]
    sim = jax.lax.dot_general(
        xtb, emb, (((1,), (0,)), ((), ())), preferred_element_type=jnp.float32
    )
    x2 = jnp.sum(xtb ** 2, axis=1).reshape(8, _T, 1)
    e2 = jnp.sum(emb ** 2, axis=0).reshape(1, 1, _NUM_EMB)
    dist = x2 + e2 - 2.0 * sim
    minval = jnp.min(dist, axis=2, keepdims=True)
    lanes = jax.lax.broadcasted_iota(jnp.int32, dist.shape, 2)
    idx = jnp.min(jnp.where(dist == minval, lanes, _NUM_EMB), axis=2)
    idx_ref[:] = idx.reshape(1, 8, _T)


def _indices(xt, embeddings):
    return pl.pallas_call(
        _argmin_body,
        grid=(_B // 8,),
        in_specs=[
            pl.BlockSpec((8, _DIM, _T), lambda i: (i, 0, 0)),
            pl.BlockSpec((_DIM, _NUM_EMB), lambda i: (0, 0)),
        ],
        out_specs=pl.BlockSpec((1, 8, _T), lambda i: (i, 0, 0)),
        out_shape=jax.ShapeDtypeStruct((_B // 8, 8, _T), jnp.int32),
    )(xt, embeddings)


def _make_gather():
    info = plsc.get_sparse_core_info()
    nc, ns = info.num_cores, info.num_subcores
    nw = nc * ns
    b_per_w = _N // nw
    mesh = plsc.VectorSubcoreMesh(core_axis_name="c", subcore_axis_name="s")

    @functools.partial(
        pl.kernel,
        mesh=mesh,
        out_type=jax.ShapeDtypeStruct((_N, 2 * _DIM), jnp.float32),
        scratch_types=[
            pltpu.VMEM((b_per_w,), jnp.int32),
            pltpu.VMEM((640, 2 * _DIM), jnp.float32),
            pltpu.SemaphoreType.DMA,
        ],
    )
    def gather(table_hbm, idx_hbm, out_hbm, idx_v, rows_v, sem):
        wid = lax.axis_index("s") * nc + lax.axis_index("c")
        base = wid * b_per_w
        pltpu.sync_copy(idx_hbm.at[pl.ds(base, b_per_w)], idx_v)
        for lo, n in ((0, 4), (4, 5)):
            copies = []
            for j in range(n):
                copies.append(
                    pltpu.async_copy(
                        table_hbm.at[idx_v.at[pl.ds((lo + j) * 128, 128)]],
                        rows_v.at[pl.ds(j * 128, 128)],
                        sem,
                    )
                )
            for c in copies:
                c.wait()
            pltpu.sync_copy(
                rows_v.at[pl.ds(0, n * 128)],
                out_hbm.at[pl.ds(base + lo * 128, n * 128)],
            )

    return gather


_gather = _make_gather()


def _detranspose_body(q_ref, out_ref):
    qb = q_ref[:, : _DIM]
    out_ref[:] = qb.T.reshape(1, _DIM, _T)


def _detranspose(q):
    return pl.pallas_call(
        _detranspose_body,
        grid=(_B,),
        in_specs=[pl.BlockSpec((_T, 2 * _DIM), lambda i: (i, 0))],
        out_specs=pl.BlockSpec((1, _DIM, _T), lambda i: (i, 0, 0)),
        out_shape=jax.ShapeDtypeStruct((_B, _DIM, _T), jnp.float32),
    )(q)


def kernel(x, embeddings):
    xt = jnp.swapaxes(x, 1, 2)
    idx = _indices(xt, embeddings).reshape(_N)
    table = jnp.pad(embeddings.T, ((0, 0), (0, _DIM)))
    q = _gather(table, idx)
    out_t = _detranspose(q)
    return jnp.swapaxes(out_t, 1, 2)


# speedup vs baseline: 1.3054x; 1.3054x over previous
import functools

import jax
import jax.numpy as jnp
from jax import lax
from jax.experimental import pallas as pl
from jax.experimental.pallas import tpu as pltpu
from jax.experimental.pallas import tpu_sc as plsc

_NUM_EMB = 1024
_DIM = 64
_B = 64
_T = 576
_N = _B * _T


def _argmin_body(xt_ref, emb_ref, idx_ref):
    xtb = xt_ref[:]
    emb = emb_ref[:]
    simn2 = jax.lax.dot_general(
        xtb * -2.0, emb, (((1,), (0,)), ((), ())),
        preferred_element_type=jnp.float32,
    )
    x2 = jnp.sum(xtb ** 2, axis=1).reshape(8, _T, 1)
    e2 = jnp.sum(emb ** 2, axis=0).reshape(1, 1, _NUM_EMB)
    dist = (x2 + e2) + simn2
    minval = jnp.min(dist, axis=2, keepdims=True)
    lanes = jax.lax.broadcasted_iota(jnp.int32, dist.shape, 2)
    idx = jnp.min(jnp.where(dist == minval, lanes, _NUM_EMB), axis=2)
    idx_ref[:] = idx.reshape(1, 8, _T)


def _indices(xt, embeddings):
    return pl.pallas_call(
        _argmin_body,
        grid=(_B // 8,),
        in_specs=[
            pl.BlockSpec((8, _DIM, _T), lambda i: (i, 0, 0)),
            pl.BlockSpec((_DIM, _NUM_EMB), lambda i: (0, 0)),
        ],
        out_specs=pl.BlockSpec((1, 8, _T), lambda i: (i, 0, 0)),
        out_shape=jax.ShapeDtypeStruct((_B // 8, 8, _T), jnp.int32),
    )(xt, embeddings)


def _make_gather():
    info = plsc.get_sparse_core_info()
    nc, ns = info.num_cores, info.num_subcores
    nw = nc * ns
    b_per_w = _N // nw
    mesh = plsc.VectorSubcoreMesh(core_axis_name="c", subcore_axis_name="s")

    @functools.partial(
        pl.kernel,
        mesh=mesh,
        out_type=jax.ShapeDtypeStruct((_N, 2 * _DIM), jnp.float32),
        scratch_types=[
            pltpu.VMEM((b_per_w,), jnp.int32),
            pltpu.VMEM((640, 2 * _DIM), jnp.float32),
            pltpu.SemaphoreType.DMA,
        ],
    )
    def gather(table_hbm, idx_hbm, out_hbm, idx_v, rows_v, sem):
        wid = lax.axis_index("s") * nc + lax.axis_index("c")
        base = wid * b_per_w
        pltpu.sync_copy(idx_hbm.at[pl.ds(base, b_per_w)], idx_v)
        for lo, n in ((0, 4), (4, 5)):
            copies = []
            for j in range(n):
                copies.append(
                    pltpu.async_copy(
                        table_hbm.at[idx_v.at[pl.ds((lo + j) * 128, 128)]],
                        rows_v.at[pl.ds(j * 128, 128)],
                        sem,
                    )
                )
            for c in copies:
                c.wait()
            pltpu.sync_copy(
                rows_v.at[pl.ds(0, n * 128)],
                out_hbm.at[pl.ds(base + lo * 128, n * 128)],
            )

    return gather


_gather = _make_gather()


def _detranspose_body(q_ref, out_ref):
    qb = q_ref[:].reshape(8, _T, 2 * _DIM)[:, :, : _DIM]
    out_ref[:] = jnp.swapaxes(qb, 1, 2)


def _detranspose(q):
    return pl.pallas_call(
        _detranspose_body,
        grid=(_B // 8,),
        in_specs=[pl.BlockSpec((8 * _T, 2 * _DIM), lambda i: (i, 0))],
        out_specs=pl.BlockSpec((8, _DIM, _T), lambda i: (i, 0, 0)),
        out_shape=jax.ShapeDtypeStruct((_B, _DIM, _T), jnp.float32),
    )(q)


def kernel(x, embeddings):
    xt = jnp.swapaxes(x, 1, 2)
    idx = _indices(xt, embeddings).reshape(_N)
    table = jnp.pad(embeddings.T, ((0, 0), (0, _DIM)))
    q = _gather(table, idx)
    out_t = _detranspose(q)
    return jnp.swapaxes(out_t, 1, 2)


# speedup vs baseline: 1.4003x; 1.0727x over previous
import functools

import jax
import jax.numpy as jnp
from jax import lax
from jax.experimental import pallas as pl
from jax.experimental.pallas import tpu as pltpu
from jax.experimental.pallas import tpu_sc as plsc

_NUM_EMB = 1024
_DIM = 64
_B = 64
_T = 576
_N = _B * _T
_SLABS = 2
_SB = _B // _SLABS
_SN = _SB * _T


def _argmin_body(xt_ref, emb_ref, idx_ref):
    xtb = xt_ref[:]
    emb = emb_ref[:]
    simn2 = jax.lax.dot_general(
        xtb * -2.0, emb, (((1,), (0,)), ((), ())),
        preferred_element_type=jnp.float32,
    )
    x2 = jnp.sum(xtb ** 2, axis=1).reshape(8, _T, 1)
    e2 = jnp.sum(emb ** 2, axis=0).reshape(1, 1, _NUM_EMB)
    dist = (x2 + e2) + simn2
    minval = jnp.min(dist, axis=2, keepdims=True)
    lanes = jax.lax.broadcasted_iota(jnp.int32, dist.shape, 2)
    idx = jnp.min(jnp.where(dist == minval, lanes, _NUM_EMB), axis=2)
    idx_ref[:] = idx.reshape(1, 8, _T)


def _indices(xt, embeddings, slab):
    blocks = _SB // 8
    return pl.pallas_call(
        _argmin_body,
        grid=(blocks,),
        in_specs=[
            pl.BlockSpec((8, _DIM, _T), lambda i: (i + slab * blocks, 0, 0)),
            pl.BlockSpec((_DIM, _NUM_EMB), lambda i: (0, 0)),
        ],
        out_specs=pl.BlockSpec((1, 8, _T), lambda i: (i, 0, 0)),
        out_shape=jax.ShapeDtypeStruct((blocks, 8, _T), jnp.int32),
    )(xt, embeddings)


def _make_gather():
    info = plsc.get_sparse_core_info()
    nc, ns = info.num_cores, info.num_subcores
    nw = nc * ns
    b_per_w = _SN // nw
    chunk = 96
    n_chunks = b_per_w // chunk

    mesh = plsc.VectorSubcoreMesh(core_axis_name="c", subcore_axis_name="s")

    @functools.partial(
        pl.kernel,
        mesh=mesh,
        out_type=jax.ShapeDtypeStruct((_SN, 2 * _DIM), jnp.float32),
        scratch_types=[
            pltpu.VMEM((b_per_w,), jnp.int32),
            pltpu.VMEM((b_per_w, 2 * _DIM), jnp.float32),
            pltpu.SemaphoreType.DMA,
        ],
    )
    def gather(table_hbm, idx_hbm, out_hbm, idx_v, rows_v, sem):
        wid = lax.axis_index("s") * nc + lax.axis_index("c")
        base = wid * b_per_w
        pltpu.sync_copy(idx_hbm.at[pl.ds(base, b_per_w)], idx_v)
        copies = []
        for j in range(n_chunks):
            copies.append(
                pltpu.async_copy(
                    table_hbm.at[idx_v.at[pl.ds(j * chunk, chunk)]],
                    rows_v.at[pl.ds(j * chunk, chunk)],
                    sem,
                )
            )
        for c in copies:
            c.wait()
        pltpu.sync_copy(rows_v, out_hbm.at[pl.ds(base, b_per_w)])

    return gather


_gather = _make_gather()


def _detrans_block(q_ref):
    return jnp.swapaxes(q_ref[:].reshape(8, _T, 2 * _DIM)[:, :, : _DIM], 1, 2)


def _detranspose_body(q0_ref, q1_ref, out_ref):
    i = pl.program_id(0)

    @pl.when(i < _SB // 8)
    def _():
        out_ref[:] = _detrans_block(q0_ref)

    @pl.when(i >= _SB // 8)
    def _():
        out_ref[:] = _detrans_block(q1_ref)


def _detranspose(q0, q1):
    blocks = _SB // 8
    return pl.pallas_call(
        _detranspose_body,
        grid=(2 * blocks,),
        in_specs=[
            pl.BlockSpec(
                (8 * _T, 2 * _DIM),
                lambda i: (jnp.minimum(i, blocks - 1), 0),
            ),
            pl.BlockSpec(
                (8 * _T, 2 * _DIM),
                lambda i: (jnp.maximum(i, blocks) - blocks, 0),
            ),
        ],
        out_specs=pl.BlockSpec((8, _DIM, _T), lambda i: (i, 0, 0)),
        out_shape=jax.ShapeDtypeStruct((_B, _DIM, _T), jnp.float32),
    )(q0, q1)


def kernel(x, embeddings):
    xt = jnp.swapaxes(x, 1, 2)
    table = jnp.pad(embeddings.T, ((0, 0), (0, _DIM)))
    idx0 = _indices(xt, embeddings, 0).reshape(_SN)
    q0 = _gather(table, idx0)
    idx1 = _indices(xt, embeddings, 1).reshape(_SN)
    q1 = _gather(table, idx1)
    out_t = _detranspose(q0, q1)
    return jnp.swapaxes(out_t, 1, 2)
